# baseline (speedup 1.0000x reference)
import functools

import jax
import jax.numpy as jnp
from jax import lax
from jax.experimental import pallas as pl
from jax.experimental.pallas import tpu as pltpu
from jax.experimental.pallas import tpu_sc as plsc

NC = 2
NS = 16
NW = NC * NS
K = 128


def _make_spmm(n_nodes, d, n_chunks, acc_rows):
    del n_nodes
    rows_per_tile = acc_rows // NS
    mesh = plsc.VectorSubcoreMesh(core_axis_name="c", subcore_axis_name="s")

    @functools.partial(
        pl.kernel,
        out_type=jax.ShapeDtypeStruct((NC, acc_rows, d), jnp.float32),
        mesh=mesh,
        scratch_types=[
            pltpu.VMEM((n_chunks // 2, K), jnp.int32),
            pltpu.VMEM((n_chunks // 2, K), jnp.int32),
            pltpu.VMEM((K, d), jnp.float32),
            pltpu.VMEM((K, d), jnp.float32),
            pltpu.VMEM_SHARED((acc_rows, d), jnp.float32),
            pltpu.SemaphoreType.DMA,
            pltpu.SemaphoreType.DMA,
            pltpu.SemaphoreType.DMA,
            pltpu.SemaphoreType.DMA,
        ],
    )
    def spmm(h_hbm, src_hbm, dst_hbm, zeros_hbm, out_hbm,
             src_v, dst_v, rows0, rows1, acc, sg0, sg1, ss0, ss1):
        c = lax.axis_index("c")
        s = lax.axis_index("s")
        wid = s * NC + c
        half = n_chunks // 2
        npair = half // 2

        pltpu.sync_copy(zeros_hbm, rows0)
        for z in range(rows_per_tile // K):
            pltpu.async_copy(rows0,
                             acc.at[pl.ds(s * rows_per_tile + z * K, K)], ss0)
        pltpu.async_copy(src_hbm.at[wid].at[pl.ds(0, half)], src_v, sg0)
        pltpu.async_copy(dst_hbm.at[wid].at[pl.ds(0, half)], dst_v, sg1)
        for z in range(rows_per_tile // K):
            pltpu.make_async_copy(
                rows0, acc.at[pl.ds(s * rows_per_tile + z * K, K)], ss0).wait()
        pltpu.make_async_copy(src_hbm.at[wid].at[pl.ds(0, half)], src_v,
                              sg0).wait()
        pltpu.make_async_copy(dst_hbm.at[wid].at[pl.ds(0, half)], dst_v,
                              sg1).wait()
        plsc.subcore_barrier()

        for h in range(2):
            if h == 1:
                pltpu.sync_copy(src_hbm.at[wid].at[pl.ds(half, half)], src_v)
                pltpu.sync_copy(dst_hbm.at[wid].at[pl.ds(half, half)], dst_v)
            pltpu.async_copy(h_hbm.at[dst_v.at[0]], rows0, sg0)
            pltpu.async_copy(h_hbm.at[dst_v.at[1]], rows1, sg1)

            @pl.loop(0, npair)
            def _pair(g):
                j0 = 2 * g
                pltpu.make_async_copy(h_hbm.at[dst_v.at[j0]], rows0, sg0).wait()
                pltpu.async_copy(rows0, acc.at[src_v.at[j0]], ss0, add=True)
                pltpu.make_async_copy(h_hbm.at[dst_v.at[j0 + 1]], rows1,
                                      sg1).wait()
                pltpu.async_copy(rows1, acc.at[src_v.at[j0 + 1]], ss1, add=True)

                pltpu.make_async_copy(rows0, acc.at[src_v.at[j0]], ss0).wait()
                pltpu.make_async_copy(rows1, acc.at[src_v.at[j0 + 1]],
                                      ss1).wait()

                @pl.when(g < npair - 1)
                def _():
                    pltpu.async_copy(h_hbm.at[dst_v.at[j0 + 2]], rows0, sg0)
                    pltpu.async_copy(h_hbm.at[dst_v.at[j0 + 3]], rows1, sg1)

        plsc.subcore_barrier()
        pltpu.sync_copy(acc.at[pl.ds(s * rows_per_tile, rows_per_tile)],
                        out_hbm.at[c].at[pl.ds(s * rows_per_tile, rows_per_tile)])

    return spmm


def _mm_relu_body(x_ref, w_ref, b_ref, o_ref):
    y = jnp.dot(x_ref[...], w_ref[...], preferred_element_type=jnp.float32)
    o_ref[...] = jnp.maximum(y + b_ref[...], 0.0)


def _sum_mm_relu_body(sa_ref, sb_ref, w_ref, b_ref, o_ref):
    x = sa_ref[0] + sb_ref[0]
    y = jnp.dot(x, w_ref[...], preferred_element_type=jnp.float32)
    o_ref[...] = jnp.maximum(y + b_ref[...], 0.0)


def _sum_mm_body(sa_ref, sb_ref, w_ref, b_ref, o_ref):
    x = sa_ref[0] + sb_ref[0]
    y = jnp.dot(x, w_ref[...], preferred_element_type=jnp.float32)
    o_ref[...] = y + b_ref[...]


def _full(shape):
    return pl.BlockSpec(shape, lambda i: (0, 0))


def _rows(bn, dcols):
    return pl.BlockSpec((bn, dcols), lambda i: (i, 0))


def kernel(node_features, edge_index, rank_mapping, W1, b1, W2, b2, Wfc, bfc):
    n = node_features.shape[1]
    d_in = node_features.shape[2]
    d_hid = W1.shape[1]
    n_cls = Wfc.shape[1]
    e = edge_index.shape[2]

    n_chunks = -(-e // (NW * K))
    n_chunks = ((n_chunks + 3) // 4) * 4
    e_pad = NW * n_chunks * K
    acc_rows = ((n + NS * K - 1) // (NS * K)) * NS * K

    src = edge_index[0, 0, :]
    dst = edge_index[0, 1, :]
    pad = e_pad - e
    pad_iota = jnp.arange(pad, dtype=jnp.int32)
    src_p = jnp.concatenate([src, n + pad_iota % (acc_rows - n)])
    dst_p = jnp.concatenate([dst, pad_iota % n])
    src_p = src_p.reshape(NW, n_chunks, K)
    dst_p = dst_p.reshape(NW, n_chunks, K)

    x = node_features[0]
    bn = 1000
    grid = (n // bn,)

    h1 = pl.pallas_call(
        _mm_relu_body,
        grid=grid,
        in_specs=[_rows(bn, d_in), _full((d_in, d_hid)), _full((1, d_hid))],
        out_specs=_rows(bn, d_hid),
        out_shape=jax.ShapeDtypeStruct((n, d_hid), jnp.float32),
    )(x, W1, b1.reshape(1, d_hid))

    zeros_h = jnp.zeros((K, d_hid), jnp.float32)
    s1 = _make_spmm(n, d_hid, n_chunks, acc_rows)(h1, src_p, dst_p, zeros_h)

    def _core_rows(cc):
        return pl.BlockSpec((1, bn, d_hid), lambda i, _c=cc: (_c, i, 0))

    h2 = pl.pallas_call(
        _sum_mm_relu_body,
        grid=grid,
        in_specs=[_core_rows(0), _core_rows(1),
                  _full((d_hid, d_hid)), _full((1, d_hid))],
        out_specs=_rows(bn, d_hid),
        out_shape=jax.ShapeDtypeStruct((n, d_hid), jnp.float32),
    )(s1, s1, W2, b2.reshape(1, d_hid))

    s2 = _make_spmm(n, d_hid, n_chunks, acc_rows)(h2, src_p, dst_p, zeros_h)

    out = pl.pallas_call(
        _sum_mm_body,
        grid=grid,
        in_specs=[_core_rows(0), _core_rows(1),
                  _full((d_hid, n_cls)), _full((1, n_cls))],
        out_specs=_rows(bn, n_cls),
        out_shape=jax.ShapeDtypeStruct((n, n_cls), jnp.float32),
    )(s2, s2, Wfc, bfc.reshape(1, n_cls))

    return out[None]


# speedup vs baseline: 1.0148x; 1.0148x over previous
import functools

import jax
import jax.numpy as jnp
from jax import lax
from jax.experimental import pallas as pl
from jax.experimental.pallas import tpu as pltpu
from jax.experimental.pallas import tpu_sc as plsc

NC = 2
NS = 16
NW = NC * NS
K = 128


def _make_spmm(n_nodes, d, n_chunks, acc_rows):
    del n_nodes
    rows_per_tile = acc_rows // NS
    mesh = plsc.VectorSubcoreMesh(core_axis_name="c", subcore_axis_name="s")

    @functools.partial(
        pl.kernel,
        out_type=jax.ShapeDtypeStruct((NC, acc_rows, d), jnp.float32),
        mesh=mesh,
        scratch_types=[
            pltpu.VMEM((n_chunks // 2, K), jnp.int32),
            pltpu.VMEM((n_chunks // 2, K), jnp.int32),
            pltpu.VMEM((K, d), jnp.float32),
            pltpu.VMEM((K, d), jnp.float32),
            pltpu.VMEM_SHARED((acc_rows, d), jnp.float32),
            pltpu.SemaphoreType.DMA,
            pltpu.SemaphoreType.DMA,
            pltpu.SemaphoreType.DMA,
            pltpu.SemaphoreType.DMA,
        ],
    )
    def spmm(h_hbm, src_hbm, dst_hbm, zeros_hbm, out_hbm,
             src_v, dst_v, rows0, rows1, acc, sg0, sg1, ss0, ss1):
        c = lax.axis_index("c")
        s = lax.axis_index("s")
        wid = s * NC + c
        half = n_chunks // 2
        npair = half // 2

        pltpu.sync_copy(zeros_hbm, rows0)
        for z in range(rows_per_tile // K):
            pltpu.async_copy(rows0,
                             acc.at[pl.ds(s * rows_per_tile + z * K, K)], ss0)
        pltpu.async_copy(src_hbm.at[wid].at[pl.ds(0, half)], src_v, sg0)
        pltpu.async_copy(dst_hbm.at[wid].at[pl.ds(0, half)], dst_v, sg1)
        for z in range(rows_per_tile // K):
            pltpu.make_async_copy(
                rows0, acc.at[pl.ds(s * rows_per_tile + z * K, K)], ss0).wait()
        pltpu.make_async_copy(src_hbm.at[wid].at[pl.ds(0, half)], src_v,
                              sg0).wait()
        pltpu.make_async_copy(dst_hbm.at[wid].at[pl.ds(0, half)], dst_v,
                              sg1).wait()
        plsc.subcore_barrier()

        for h in range(2):
            if h == 1:
                pltpu.sync_copy(src_hbm.at[wid].at[pl.ds(half, half)], src_v)
                pltpu.sync_copy(dst_hbm.at[wid].at[pl.ds(half, half)], dst_v)
            pltpu.async_copy(h_hbm.at[dst_v.at[0]], rows0, sg0)
            pltpu.async_copy(h_hbm.at[dst_v.at[1]], rows1, sg1)

            @pl.loop(0, npair)
            def _pair(g):
                j0 = 2 * g
                pltpu.make_async_copy(h_hbm.at[dst_v.at[j0]], rows0, sg0).wait()
                pltpu.async_copy(rows0, acc.at[src_v.at[j0]], ss0, add=True)
                pltpu.make_async_copy(h_hbm.at[dst_v.at[j0 + 1]], rows1,
                                      sg1).wait()
                pltpu.async_copy(rows1, acc.at[src_v.at[j0 + 1]], ss1, add=True)

                pltpu.make_async_copy(rows0, acc.at[src_v.at[j0]], ss0).wait()

                @pl.when(g < npair - 1)
                def _():
                    pltpu.async_copy(h_hbm.at[dst_v.at[j0 + 2]], rows0, sg0)

                pltpu.make_async_copy(rows1, acc.at[src_v.at[j0 + 1]],
                                      ss1).wait()

                @pl.when(g < npair - 1)
                def _():
                    pltpu.async_copy(h_hbm.at[dst_v.at[j0 + 3]], rows1, sg1)

        plsc.subcore_barrier()
        pltpu.sync_copy(acc.at[pl.ds(s * rows_per_tile, rows_per_tile)],
                        out_hbm.at[c].at[pl.ds(s * rows_per_tile, rows_per_tile)])

    return spmm


def _mm_relu_body(x_ref, w_ref, b_ref, o_ref):
    y = jnp.dot(x_ref[...], w_ref[...], preferred_element_type=jnp.float32)
    o_ref[...] = jnp.maximum(y + b_ref[...], 0.0)


def _sum_mm_relu_body(sa_ref, sb_ref, w_ref, b_ref, o_ref):
    x = sa_ref[0] + sb_ref[0]
    y = jnp.dot(x, w_ref[...], preferred_element_type=jnp.float32)
    o_ref[...] = jnp.maximum(y + b_ref[...], 0.0)


def _sum_mm_body(sa_ref, sb_ref, w_ref, b_ref, o_ref):
    x = sa_ref[0] + sb_ref[0]
    y = jnp.dot(x, w_ref[...], preferred_element_type=jnp.float32)
    o_ref[...] = y + b_ref[...]


def _full(shape):
    return pl.BlockSpec(shape, lambda i: (0, 0))


def _rows(bn, dcols):
    return pl.BlockSpec((bn, dcols), lambda i: (i, 0))


def kernel(node_features, edge_index, rank_mapping, W1, b1, W2, b2, Wfc, bfc):
    n = node_features.shape[1]
    d_in = node_features.shape[2]
    d_hid = W1.shape[1]
    n_cls = Wfc.shape[1]
    e = edge_index.shape[2]

    n_chunks = -(-e // (NW * K))
    n_chunks = ((n_chunks + 3) // 4) * 4
    e_pad = NW * n_chunks * K
    acc_rows = ((n + NS * K - 1) // (NS * K)) * NS * K

    src = edge_index[0, 0, :]
    dst = edge_index[0, 1, :]
    pad = e_pad - e
    pad_iota = jnp.arange(pad, dtype=jnp.int32)
    src_p = jnp.concatenate([src, n + pad_iota % (acc_rows - n)])
    dst_p = jnp.concatenate([dst, pad_iota % n])
    src_p = src_p.reshape(NW, n_chunks, K)
    dst_p = dst_p.reshape(NW, n_chunks, K)

    x = node_features[0]
    bn = 1000
    grid = (n // bn,)

    h1 = pl.pallas_call(
        _mm_relu_body,
        grid=grid,
        in_specs=[_rows(bn, d_in), _full((d_in, d_hid)), _full((1, d_hid))],
        out_specs=_rows(bn, d_hid),
        out_shape=jax.ShapeDtypeStruct((n, d_hid), jnp.float32),
    )(x, W1, b1.reshape(1, d_hid))

    zeros_h = jnp.zeros((K, d_hid), jnp.float32)
    s1 = _make_spmm(n, d_hid, n_chunks, acc_rows)(h1, src_p, dst_p, zeros_h)

    def _core_rows(cc):
        return pl.BlockSpec((1, bn, d_hid), lambda i, _c=cc: (_c, i, 0))

    h2 = pl.pallas_call(
        _sum_mm_relu_body,
        grid=grid,
        in_specs=[_core_rows(0), _core_rows(1),
                  _full((d_hid, d_hid)), _full((1, d_hid))],
        out_specs=_rows(bn, d_hid),
        out_shape=jax.ShapeDtypeStruct((n, d_hid), jnp.float32),
    )(s1, s1, W2, b2.reshape(1, d_hid))

    s2 = _make_spmm(n, d_hid, n_chunks, acc_rows)(h2, src_p, dst_p, zeros_h)

    out = pl.pallas_call(
        _sum_mm_body,
        grid=grid,
        in_specs=[_core_rows(0), _core_rows(1),
                  _full((d_hid, n_cls)), _full((1, n_cls))],
        out_specs=_rows(bn, n_cls),
        out_shape=jax.ShapeDtypeStruct((n, n_cls), jnp.float32),
    )(s2, s2, Wfc, bfc.reshape(1, n_cls))

    return out[None]


# speedup vs baseline: 1.1030x; 1.0869x over previous
import functools

import jax
import jax.numpy as jnp
from jax import lax
from jax.experimental import pallas as pl
from jax.experimental.pallas import tpu as pltpu
from jax.experimental.pallas import tpu_sc as plsc

NC = 2
NS = 16
NW = NC * NS
K = 128


def _make_spmm(n_nodes, d, n_chunks, acc_rows):
    del n_nodes
    rows_per_tile = acc_rows // NS
    mesh = plsc.VectorSubcoreMesh(core_axis_name="c", subcore_axis_name="s")

    @functools.partial(
        pl.kernel,
        out_type=jax.ShapeDtypeStruct((NC, acc_rows, d), jnp.float32),
        mesh=mesh,
        scratch_types=[
            pltpu.VMEM((n_chunks // 2, K), jnp.int32),
            pltpu.VMEM((n_chunks // 2, K), jnp.int32),
            pltpu.VMEM((K, d), jnp.float32),
            pltpu.VMEM((K, d), jnp.float32),
            pltpu.VMEM_SHARED((acc_rows, d), jnp.float32),
            pltpu.SemaphoreType.DMA,
            pltpu.SemaphoreType.DMA,
        ],
    )
    def spmm(h_hbm, src_hbm, dst_hbm, zeros_hbm, out_hbm,
             src_v, dst_v, rows0, rows1, acc, sem0, sem1):
        c = lax.axis_index("c")
        s = lax.axis_index("s")
        wid = s * NC + c

        half = n_chunks // 2
        pltpu.async_copy(src_hbm.at[wid].at[pl.ds(0, half)], src_v, sem0)
        pltpu.async_copy(dst_hbm.at[wid].at[pl.ds(0, half)], dst_v, sem1)
        pltpu.sync_copy(zeros_hbm, rows0)
        for z in range(rows_per_tile // K):
            pltpu.sync_copy(rows0, acc.at[pl.ds(s * rows_per_tile + z * K, K)])
        pltpu.make_async_copy(src_hbm.at[wid].at[pl.ds(0, half)], src_v,
                              sem0).wait()
        pltpu.make_async_copy(dst_hbm.at[wid].at[pl.ds(0, half)], dst_v,
                              sem1).wait()

        plsc.subcore_barrier()

        for h in range(2):
            if h == 1:
                pltpu.sync_copy(src_hbm.at[wid].at[pl.ds(half, half)], src_v)
                pltpu.sync_copy(dst_hbm.at[wid].at[pl.ds(half, half)], dst_v)
            pltpu.async_copy(h_hbm.at[dst_v.at[0]], rows0, sem0)

            @pl.loop(0, half // 2)
            def _pair(g):
                j0 = 2 * g
                pltpu.make_async_copy(h_hbm.at[dst_v.at[j0]], rows0, sem0).wait()
                pltpu.async_copy(h_hbm.at[dst_v.at[j0 + 1]], rows1, sem1)
                pltpu.sync_copy(rows0, acc.at[src_v.at[j0]], add=True)
                pltpu.make_async_copy(h_hbm.at[dst_v.at[j0 + 1]], rows1,
                                      sem1).wait()

                @pl.when(g < half // 2 - 1)
                def _():
                    pltpu.async_copy(h_hbm.at[dst_v.at[j0 + 2]], rows0, sem0)

                pltpu.sync_copy(rows1, acc.at[src_v.at[j0 + 1]], add=True)

        plsc.subcore_barrier()
        pltpu.sync_copy(acc.at[pl.ds(s * rows_per_tile, rows_per_tile)],
                        out_hbm.at[c].at[pl.ds(s * rows_per_tile, rows_per_tile)])

    return spmm


def _mm_relu_body(x_ref, w_ref, b_ref, o_ref):
    y = jnp.dot(x_ref[...], w_ref[...], preferred_element_type=jnp.float32)
    o_ref[...] = jnp.maximum(y + b_ref[...], 0.0)


def _sum_mm_relu_body(sa_ref, sb_ref, w_ref, b_ref, o_ref):
    x = sa_ref[0] + sb_ref[0]
    y = jnp.dot(x, w_ref[...], preferred_element_type=jnp.float32)
    o_ref[...] = jnp.maximum(y + b_ref[...], 0.0)


def _sum_mm_body(sa_ref, sb_ref, w_ref, b_ref, o_ref):
    x = sa_ref[0] + sb_ref[0]
    y = jnp.dot(x, w_ref[...], preferred_element_type=jnp.float32)
    o_ref[...] = y + b_ref[...]


def _full(shape):
    return pl.BlockSpec(shape, lambda i: (0, 0))


def _rows(bn, dcols):
    return pl.BlockSpec((bn, dcols), lambda i: (i, 0))


def kernel(node_features, edge_index, rank_mapping, W1, b1, W2, b2, Wfc, bfc):
    n = node_features.shape[1]
    d_in = node_features.shape[2]
    d_hid = W1.shape[1]
    n_cls = Wfc.shape[1]
    e = edge_index.shape[2]

    n_chunks = -(-e // (NW * K))
    n_chunks = ((n_chunks + 3) // 4) * 4
    e_pad = NW * n_chunks * K
    acc_rows = ((n + NS * K - 1) // (NS * K)) * NS * K

    src = edge_index[0, 0, :]
    dst = edge_index[0, 1, :]
    pad = e_pad - e
    pad_iota = jnp.arange(pad, dtype=jnp.int32)
    src_p = jnp.concatenate([src, n + pad_iota % (acc_rows - n)])
    dst_p = jnp.concatenate([dst, pad_iota % n])
    src_p = src_p.reshape(NW, n_chunks, K)
    dst_p = dst_p.reshape(NW, n_chunks, K)

    x = node_features[0]
    bn = 1000
    grid = (n // bn,)

    h1 = pl.pallas_call(
        _mm_relu_body,
        grid=grid,
        in_specs=[_rows(bn, d_in), _full((d_in, d_hid)), _full((1, d_hid))],
        out_specs=_rows(bn, d_hid),
        out_shape=jax.ShapeDtypeStruct((n, d_hid), jnp.float32),
    )(x, W1, b1.reshape(1, d_hid))

    zeros_h = jnp.zeros((K, d_hid), jnp.float32)
    s1 = _make_spmm(n, d_hid, n_chunks, acc_rows)(h1, src_p, dst_p, zeros_h)

    def _core_rows(cc):
        return pl.BlockSpec((1, bn, d_hid), lambda i, _c=cc: (_c, i, 0))

    h2 = pl.pallas_call(
        _sum_mm_relu_body,
        grid=grid,
        in_specs=[_core_rows(0), _core_rows(1),
                  _full((d_hid, d_hid)), _full((1, d_hid))],
        out_specs=_rows(bn, d_hid),
        out_shape=jax.ShapeDtypeStruct((n, d_hid), jnp.float32),
    )(s1, s1, W2, b2.reshape(1, d_hid))

    s2 = _make_spmm(n, d_hid, n_chunks, acc_rows)(h2, src_p, dst_p, zeros_h)

    out = pl.pallas_call(
        _sum_mm_body,
        grid=grid,
        in_specs=[_core_rows(0), _core_rows(1),
                  _full((d_hid, n_cls)), _full((1, n_cls))],
        out_specs=_rows(bn, n_cls),
        out_shape=jax.ShapeDtypeStruct((n, n_cls), jnp.float32),
    )(s2, s2, Wfc, bfc.reshape(1, n_cls))

    return out[None]
